# baseline (speedup 1.0000x reference)
import jax
import jax.numpy as jnp
from jax import lax
from jax.experimental import pallas as pl
from jax.experimental.pallas import tpu as pltpu

POS_THR, NEG_THR = 0.5, 0.4
ALPHA, GAMMA = 0.25, 2.0
BBOX_W = 2.0
BNL = 4096


def _focal(x, t):
    p = jax.nn.sigmoid(x)
    p_t = jnp.where(t == 1.0, p, 1.0 - p)
    one_m = 1.0 - p_t
    w = one_m * one_m
    bce = jnp.maximum(x, 0.0) - x * t + jnp.log1p(jnp.exp(-jnp.abs(x)))
    a_t = jnp.where(t == 1.0, ALPHA, 1.0 - ALPHA)
    return a_t * w * bce


def _body(n_real, n_blocks, g, pred_ref, gtb_ref, out_ref, st_ref, acc_ref):
    j = pl.program_id(1)

    @pl.when(j == 0)
    def _init():
        st_ref[...] = jnp.zeros((g, 8), jnp.float32)
        st_ref[:, 0:1] = jnp.full((g, 1), -1.0, jnp.float32)
        st_ref[:, 1:2] = jnp.full((g, 1), 1e9, jnp.float32)
        acc_ref[...] = jnp.zeros((1, 128), jnp.float32)

    pred = pred_ref[0]
    cx = pred[0:1, :]
    cy = pred[1:2, :]
    w = pred[2:3, :]
    h = pred[3:4, :]
    obj = pred[4:5, :]
    px1 = cx - w * 0.5
    py1 = cy - h * 0.5
    px2 = cx + w * 0.5
    py2 = cy + h * 0.5
    parea = (px2 - px1) * (py2 - py1)

    gtb = gtb_ref[0]
    gx1 = gtb[:, 0:1]
    gy1 = gtb[:, 1:2]
    gx2 = gtb[:, 2:3]
    gy2 = gtb[:, 3:4]
    garea = (gx2 - gx1) * (gy2 - gy1)

    ix1 = jnp.maximum(px1, gx1)
    iy1 = jnp.maximum(py1, gy1)
    ix2 = jnp.minimum(px2, gx2)
    iy2 = jnp.minimum(py2, gy2)
    inter = jnp.maximum(ix2 - ix1, 0.0) * jnp.maximum(iy2 - iy1, 0.0)
    union = parea + garea - inter
    iou = inter / jnp.maximum(union, 1e-6)

    bgp = jnp.max(iou, axis=0, keepdims=True)
    sub_iota = lax.broadcasted_iota(jnp.int32, (g, BNL), 0)
    ridx = jnp.min(jnp.where(iou == bgp, sub_iota, g), axis=0, keepdims=True)
    selr = sub_iota == ridx
    selr_f = jnp.where(selr, 1.0, 0.0)
    matched = lax.dot_general(gtb, selr_f, (((0,), (0,)), ((), ())),
                              precision=lax.Precision.HIGHEST,
                              preferred_element_type=jnp.float32)
    mx1 = matched[0:1, :]
    my1 = matched[1:2, :]
    mx2 = matched[2:3, :]
    my2 = matched[3:4, :]

    gix1 = jnp.maximum(px1, mx1)
    giy1 = jnp.maximum(py1, my1)
    gix2 = jnp.minimum(px2, mx2)
    giy2 = jnp.minimum(py2, my2)
    ginter = jnp.maximum(gix2 - gix1, 0.0) * jnp.maximum(giy2 - giy1, 0.0)
    marea = (mx2 - mx1) * (my2 - my1)
    gunion = parea + marea - ginter
    giou_iou = ginter / jnp.maximum(gunion, 1e-6)
    ex1 = jnp.minimum(px1, mx1)
    ey1 = jnp.minimum(py1, my1)
    ex2 = jnp.maximum(px2, mx2)
    ey2 = jnp.maximum(py2, my2)
    enc = (ex2 - ex1) * (ey2 - ey1)
    giou = giou_iou - (enc - gunion) / jnp.maximum(enc, 1e-6)

    lane_loc = lax.broadcasted_iota(jnp.int32, (1, BNL), 1)
    gmask = (lane_loc + j * BNL) < n_real
    posb = bgp > POS_THR
    negb = bgp < NEG_THR
    valid = (posb | negb) & gmask
    fl = _focal(obj, jnp.where(posb, 1.0, 0.0))
    acc_ref[0:1, 0:1] += jnp.sum(jnp.where(valid, fl, 0.0), keepdims=True)
    acc_ref[0:1, 1:2] += jnp.sum(valid.astype(jnp.float32), keepdims=True)
    acc_ref[0:1, 2:3] += jnp.sum(jnp.where(posb, giou, 0.0), keepdims=True)
    acc_ref[0:1, 3:4] += jnp.sum(posb.astype(jnp.float32), keepdims=True)

    bm = jnp.max(iou, axis=1, keepdims=True)
    lane_glob = lane_loc + j * BNL
    lidx = jnp.min(jnp.where(iou == bm, lane_glob, 10 * BNL * n_blocks),
                   axis=1, keepdims=True)
    sel = lane_glob == lidx
    sel_f = jnp.where(sel, 1.0, 0.0)
    vals3 = jnp.concatenate([obj, bgp, giou], axis=0)
    payload = lax.dot_general(sel_f, vals3, (((1,), (1,)), ((), ())),
                              precision=lax.Precision.HIGHEST,
                              preferred_element_type=jnp.float32)
    p_obj = payload[:, 0:1]
    p_bgp = payload[:, 1:2]
    p_gio = payload[:, 2:3]

    cm = st_ref[:, 0:1]
    upd = bm > cm
    st_ref[:, 0:1] = jnp.where(upd, bm, cm)
    st_ref[:, 1:2] = jnp.where(upd, lidx.astype(jnp.float32), st_ref[:, 1:2])
    st_ref[:, 2:3] = jnp.where(upd, p_obj, st_ref[:, 2:3])
    st_ref[:, 3:4] = jnp.where(upd, p_bgp, st_ref[:, 3:4])
    st_ref[:, 4:5] = jnp.where(upd, p_gio, st_ref[:, 4:5])

    @pl.when(j == n_blocks - 1)
    def _fin():
        cm_c = st_ref[:, 0:1]
        cidx_c = st_ref[:, 1:2]
        pobj_c = st_ref[:, 2:3]
        pbgp_c = st_ref[:, 3:4]
        pgio_c = st_ref[:, 4:5]

        io0 = lax.broadcasted_iota(jnp.int32, (g, g), 0)
        io1 = lax.broadcasted_iota(jnp.int32, (g, g), 1)
        eye = io0 == io1

        def to_row(col):
            return jnp.sum(jnp.where(eye, col, 0.0), axis=0, keepdims=True)

        cm_r = to_row(cm_c)
        cidx_r = to_row(cidx_c)
        pobj_r = to_row(pobj_c)
        pbgp_r = to_row(pbgp_c)
        pgio_r = to_row(pgio_c)

        qual_c = cm_c > 0.1
        qual_r = cm_r > 0.1
        dup_pq = (cidx_c == cidx_r) & qual_c & (io0 < io1)
        dup_r = jnp.max(jnp.where(dup_pq, 1, 0), axis=0, keepdims=True) > 0
        keep = qual_r & jnp.logical_not(dup_r)

        already = pbgp_r > POS_THR
        newly = keep & jnp.logical_not(already)
        was_neg = pbgp_r < NEG_THR
        fl1 = _focal(pobj_r, jnp.ones((1, g), jnp.float32))
        fl0 = _focal(pobj_r, jnp.zeros((1, g), jnp.float32))
        dcls = jnp.sum(
            jnp.where(newly, fl1 - jnp.where(was_neg, fl0, 0.0), 0.0),
            keepdims=True)
        dvcnt = jnp.sum((newly & jnp.logical_not(was_neg)).astype(jnp.float32),
                        keepdims=True)
        dnpos = jnp.sum(newly.astype(jnp.float32), keepdims=True)
        dreg = jnp.sum(jnp.where(newly, pgio_r, 0.0), keepdims=True)

        cls_sum = acc_ref[0:1, 0:1] + dcls
        vcnt = acc_ref[0:1, 1:2] + dvcnt
        reg_sum = acc_ref[0:1, 2:3] + dreg
        npos = acc_ref[0:1, 3:4] + dnpos

        cls_l = jnp.where(vcnt > 0, cls_sum / jnp.maximum(vcnt, 1.0), 0.0)
        reg_l = jnp.where(npos > 0, 1.0 - reg_sum / jnp.maximum(npos, 1.0),
                          0.0)

        lane = lax.broadcasted_iota(jnp.int32, (8, 128), 1)
        row = jnp.where(lane == 0, cls_l,
                        jnp.where(lane == 1, reg_l,
                                  jnp.where(lane == 2, npos, 0.0)))
        out_ref[0] = row


def kernel(predictions, gt_boxes, gt_labels):
    b, n, _ = predictions.shape
    g = gt_boxes.shape[1]
    n_pad = ((n + BNL - 1) // BNL) * BNL
    n_blocks = n_pad // BNL

    preds_t = jnp.transpose(predictions, (0, 2, 1))
    preds_t = jnp.pad(preds_t, ((0, 0), (0, 0), (0, n_pad - n)))

    import functools
    body = functools.partial(_body, n, n_blocks, g)

    out = pl.pallas_call(
        body,
        grid=(b, n_blocks),
        in_specs=[
            pl.BlockSpec((1, 5, BNL), lambda i, j: (i, 0, j)),
            pl.BlockSpec((1, g, 4), lambda i, j: (i, 0, 0)),
        ],
        out_specs=pl.BlockSpec((1, 8, 128), lambda i, j: (i, 0, 0)),
        out_shape=jax.ShapeDtypeStruct((b, 8, 128), jnp.float32),
        scratch_shapes=[
            pltpu.VMEM((g, 8), jnp.float32),
            pltpu.VMEM((1, 128), jnp.float32),
        ],
        compiler_params=pltpu.CompilerParams(
            dimension_semantics=("arbitrary", "arbitrary"),
        ),
    )(preds_t, gt_boxes)

    cls_l = out[:, 0, 0]
    reg_l = out[:, 0, 1]
    npos = out[:, 0, 2]
    total_cls = cls_l.sum() / b
    num_pos = jnp.maximum(npos.sum(), 1.0)
    total_reg = reg_l.sum() / num_pos * b
    return total_cls + BBOX_W * total_reg


# speedup vs baseline: 1.2025x; 1.2025x over previous
import jax
import jax.numpy as jnp
from jax import lax
from jax.experimental import pallas as pl
from jax.experimental.pallas import tpu as pltpu

POS_THR, NEG_THR = 0.5, 0.4
ALPHA, GAMMA = 0.25, 2.0
BBOX_W = 2.0
BNL = 4096


def _focal(x, t):
    p = jax.nn.sigmoid(x)
    p_t = jnp.where(t == 1.0, p, 1.0 - p)
    one_m = 1.0 - p_t
    w = one_m * one_m
    bce = jnp.maximum(x, 0.0) - x * t + jnp.log1p(jnp.exp(-jnp.abs(x)))
    a_t = jnp.where(t == 1.0, ALPHA, 1.0 - ALPHA)
    return a_t * w * bce


def _body(n_real, n_blocks, g, pred_ref, gtb_ref, out_ref, st_ref, acc_ref):
    j = pl.program_id(1)

    @pl.when(j == 0)
    def _init():
        st_ref[...] = jnp.zeros((g, 8), jnp.float32)
        st_ref[:, 0:1] = jnp.full((g, 1), -1.0, jnp.float32)
        st_ref[:, 1:2] = jnp.full((g, 1), 1e9, jnp.float32)
        acc_ref[...] = jnp.zeros((1, 128), jnp.float32)

    pred = pred_ref[0]
    cx = pred[0:1, :]
    cy = pred[1:2, :]
    w = pred[2:3, :]
    h = pred[3:4, :]
    obj = pred[4:5, :]
    px1 = cx - w * 0.5
    py1 = cy - h * 0.5
    px2 = cx + w * 0.5
    py2 = cy + h * 0.5
    parea = (px2 - px1) * (py2 - py1)

    gtb = gtb_ref[0]
    gx1 = gtb[:, 0:1]
    gy1 = gtb[:, 1:2]
    gx2 = gtb[:, 2:3]
    gy2 = gtb[:, 3:4]
    garea = (gx2 - gx1) * (gy2 - gy1)

    ix1 = jnp.maximum(px1, gx1)
    iy1 = jnp.maximum(py1, gy1)
    ix2 = jnp.minimum(px2, gx2)
    iy2 = jnp.minimum(py2, gy2)
    inter = jnp.maximum(ix2 - ix1, 0.0) * jnp.maximum(iy2 - iy1, 0.0)
    union = parea + garea - inter
    iou = inter / jnp.maximum(union, 1e-6)

    bgp = jnp.max(iou, axis=0, keepdims=True)
    sub_iota = lax.broadcasted_iota(jnp.int32, (g, BNL), 0)
    ridx = jnp.min(jnp.where(iou == bgp, sub_iota, g), axis=0, keepdims=True)
    selr = sub_iota == ridx
    selr_f = jnp.where(selr, 1.0, 0.0)
    g_hi = gtb.astype(jnp.bfloat16).astype(jnp.float32)
    g_r1 = gtb - g_hi
    g_md = g_r1.astype(jnp.bfloat16).astype(jnp.float32)
    g_lo = g_r1 - g_md

    def _sel_mm(m):
        return lax.dot_general(m, selr_f, (((0,), (0,)), ((), ())),
                               preferred_element_type=jnp.float32)

    matched = (_sel_mm(g_hi) + _sel_mm(g_md)) + _sel_mm(g_lo)
    mx1 = matched[0:1, :]
    my1 = matched[1:2, :]
    mx2 = matched[2:3, :]
    my2 = matched[3:4, :]

    gix1 = jnp.maximum(px1, mx1)
    giy1 = jnp.maximum(py1, my1)
    gix2 = jnp.minimum(px2, mx2)
    giy2 = jnp.minimum(py2, my2)
    ginter = jnp.maximum(gix2 - gix1, 0.0) * jnp.maximum(giy2 - giy1, 0.0)
    marea = (mx2 - mx1) * (my2 - my1)
    gunion = parea + marea - ginter
    giou_iou = ginter / jnp.maximum(gunion, 1e-6)
    ex1 = jnp.minimum(px1, mx1)
    ey1 = jnp.minimum(py1, my1)
    ex2 = jnp.maximum(px2, mx2)
    ey2 = jnp.maximum(py2, my2)
    enc = (ex2 - ex1) * (ey2 - ey1)
    giou = giou_iou - (enc - gunion) / jnp.maximum(enc, 1e-6)

    lane_loc = lax.broadcasted_iota(jnp.int32, (1, BNL), 1)
    gmask = (lane_loc + j * BNL) < n_real
    posb = bgp > POS_THR
    negb = bgp < NEG_THR
    valid = (posb | negb) & gmask
    fl = _focal(obj, jnp.where(posb, 1.0, 0.0))
    acc_ref[0:1, 0:1] += jnp.sum(jnp.where(valid, fl, 0.0), keepdims=True)
    acc_ref[0:1, 1:2] += jnp.sum(valid.astype(jnp.float32), keepdims=True)
    acc_ref[0:1, 2:3] += jnp.sum(jnp.where(posb, giou, 0.0), keepdims=True)
    acc_ref[0:1, 3:4] += jnp.sum(posb.astype(jnp.float32), keepdims=True)

    bm = jnp.max(iou, axis=1, keepdims=True)
    lane_glob = lane_loc + j * BNL
    lidx = jnp.min(jnp.where(iou == bm, lane_glob, 10 * BNL * n_blocks),
                   axis=1, keepdims=True)
    sel = lane_glob == lidx
    p_obj = jnp.sum(jnp.where(sel, obj, 0.0), axis=1, keepdims=True)
    p_bgp = jnp.sum(jnp.where(sel, bgp, 0.0), axis=1, keepdims=True)
    p_gio = jnp.sum(jnp.where(sel, giou, 0.0), axis=1, keepdims=True)

    cm = st_ref[:, 0:1]
    upd = bm > cm
    st_ref[:, 0:1] = jnp.where(upd, bm, cm)
    st_ref[:, 1:2] = jnp.where(upd, lidx.astype(jnp.float32), st_ref[:, 1:2])
    st_ref[:, 2:3] = jnp.where(upd, p_obj, st_ref[:, 2:3])
    st_ref[:, 3:4] = jnp.where(upd, p_bgp, st_ref[:, 3:4])
    st_ref[:, 4:5] = jnp.where(upd, p_gio, st_ref[:, 4:5])

    @pl.when(j == n_blocks - 1)
    def _fin():
        cm_c = st_ref[:, 0:1]
        cidx_c = st_ref[:, 1:2]
        pobj_c = st_ref[:, 2:3]
        pbgp_c = st_ref[:, 3:4]
        pgio_c = st_ref[:, 4:5]

        io0 = lax.broadcasted_iota(jnp.int32, (g, g), 0)
        io1 = lax.broadcasted_iota(jnp.int32, (g, g), 1)
        eye = io0 == io1

        def to_row(col):
            return jnp.sum(jnp.where(eye, col, 0.0), axis=0, keepdims=True)

        cm_r = to_row(cm_c)
        cidx_r = to_row(cidx_c)
        pobj_r = to_row(pobj_c)
        pbgp_r = to_row(pbgp_c)
        pgio_r = to_row(pgio_c)

        qual_c = cm_c > 0.1
        qual_r = cm_r > 0.1
        dup_pq = (cidx_c == cidx_r) & qual_c & (io0 < io1)
        dup_r = jnp.max(jnp.where(dup_pq, 1, 0), axis=0, keepdims=True) > 0
        keep = qual_r & jnp.logical_not(dup_r)

        already = pbgp_r > POS_THR
        newly = keep & jnp.logical_not(already)
        was_neg = pbgp_r < NEG_THR
        fl1 = _focal(pobj_r, jnp.ones((1, g), jnp.float32))
        fl0 = _focal(pobj_r, jnp.zeros((1, g), jnp.float32))
        dcls = jnp.sum(
            jnp.where(newly, fl1 - jnp.where(was_neg, fl0, 0.0), 0.0),
            keepdims=True)
        dvcnt = jnp.sum((newly & jnp.logical_not(was_neg)).astype(jnp.float32),
                        keepdims=True)
        dnpos = jnp.sum(newly.astype(jnp.float32), keepdims=True)
        dreg = jnp.sum(jnp.where(newly, pgio_r, 0.0), keepdims=True)

        cls_sum = acc_ref[0:1, 0:1] + dcls
        vcnt = acc_ref[0:1, 1:2] + dvcnt
        reg_sum = acc_ref[0:1, 2:3] + dreg
        npos = acc_ref[0:1, 3:4] + dnpos

        cls_l = jnp.where(vcnt > 0, cls_sum / jnp.maximum(vcnt, 1.0), 0.0)
        reg_l = jnp.where(npos > 0, 1.0 - reg_sum / jnp.maximum(npos, 1.0),
                          0.0)

        lane = lax.broadcasted_iota(jnp.int32, (8, 128), 1)
        row = jnp.where(lane == 0, cls_l,
                        jnp.where(lane == 1, reg_l,
                                  jnp.where(lane == 2, npos, 0.0)))
        out_ref[0] = row


def kernel(predictions, gt_boxes, gt_labels):
    b, n, _ = predictions.shape
    g = gt_boxes.shape[1]
    n_pad = ((n + BNL - 1) // BNL) * BNL
    n_blocks = n_pad // BNL

    preds_t = jnp.transpose(predictions, (0, 2, 1))
    preds_t = jnp.pad(preds_t, ((0, 0), (0, 0), (0, n_pad - n)))

    import functools
    body = functools.partial(_body, n, n_blocks, g)

    out = pl.pallas_call(
        body,
        grid=(b, n_blocks),
        in_specs=[
            pl.BlockSpec((1, 5, BNL), lambda i, j: (i, 0, j)),
            pl.BlockSpec((1, g, 4), lambda i, j: (i, 0, 0)),
        ],
        out_specs=pl.BlockSpec((1, 8, 128), lambda i, j: (i, 0, 0)),
        out_shape=jax.ShapeDtypeStruct((b, 8, 128), jnp.float32),
        scratch_shapes=[
            pltpu.VMEM((g, 8), jnp.float32),
            pltpu.VMEM((1, 128), jnp.float32),
        ],
        compiler_params=pltpu.CompilerParams(
            dimension_semantics=("arbitrary", "arbitrary"),
        ),
    )(preds_t, gt_boxes)

    cls_l = out[:, 0, 0]
    reg_l = out[:, 0, 1]
    npos = out[:, 0, 2]
    total_cls = cls_l.sum() / b
    num_pos = jnp.maximum(npos.sum(), 1.0)
    total_reg = reg_l.sum() / num_pos * b
    return total_cls + BBOX_W * total_reg
